# Initial kernel scaffold; baseline (speedup 1.0000x reference)
import jax
import jax.numpy as jnp
from jax.experimental import pallas as pl


def kernel(f_atoms, f_bonds, W_i, W_h, W_o, b_o, bn_w, bn_b, a2b, b2a, b2revb):
    raise NotImplementedError("write your pallas kernel here")


# speedup vs baseline: 1.5935x; 1.5935x over previous
import functools

import jax
import jax.numpy as jnp
from jax import lax
from jax.experimental import pallas as pl
from jax.experimental.pallas import tpu as pltpu
from jax.experimental.pallas import tpu_sc as plsc

E = 320001
NA = 10001
NB = 32
H = 128
AF = 133
BF = 147
EPS = 1e-5

NW = 32
A_PAD = 10240
PW_A = A_PAD // NW
CA = 8
SA = PW_A // CA
E_PAD = 321536
PW_B = E_PAD // NW
CB = 64
SB = PW_B // CB

NPAIR = 160000
PB = 1000
E2 = 320002


def _sc_mesh():
    return plsc.VectorSubcoreMesh(core_axis_name="c", subcore_axis_name="s")


def _gather_sum_sc(src, a2b_flat, do_relu):
    @functools.partial(
        pl.kernel,
        out_type=jax.ShapeDtypeStruct((A_PAD, H), jnp.float32),
        mesh=_sc_mesh(),
        scratch_types=[
            pltpu.VMEM((PW_A * NB,), jnp.int32),
            pltpu.VMEM((CA * NB, H), jnp.float32),
            pltpu.VMEM((CA, H), jnp.float32),
            pltpu.SemaphoreType.DMA,
        ],
    )
    def k(src_ref, idx_hbm, out_ref, idx_v, rows_v, acc_v, sem):
        w = lax.axis_index("s") * 2 + lax.axis_index("c")
        pltpu.sync_copy(idx_hbm.at[pl.ds(w * (PW_A * NB), PW_A * NB)], idx_v)

        def step(s, carry):
            cp = pltpu.make_async_copy(
                src_ref.at[idx_v.at[pl.ds(s * (CA * NB), CA * NB)]], rows_v, sem)
            cp.start()
            cp.wait()

            def atom(a, c):
                for j in range(H // 16):
                    acc = rows_v[a * NB, pl.ds(j * 16, 16)]
                    if do_relu:
                        acc = jnp.maximum(acc, 0.0)
                    for kk in range(1, NB):
                        v = rows_v[a * NB + kk, pl.ds(j * 16, 16)]
                        if do_relu:
                            v = jnp.maximum(v, 0.0)
                        acc = acc + v
                    acc_v[a, pl.ds(j * 16, 16)] = acc
                return c

            lax.fori_loop(0, CA, atom, 0)
            pltpu.sync_copy(acc_v, out_ref.at[pl.ds(w * PW_A + s * CA, CA)])
            return carry

        lax.fori_loop(0, SA, step, 0)

    return k(src, a2b_flat)


def _pre_sc(a_tab, src, b2a_pad, b2revb_pad, do_relu):
    @functools.partial(
        pl.kernel,
        out_type=jax.ShapeDtypeStruct((E_PAD, H), jnp.float32),
        mesh=_sc_mesh(),
        scratch_types=[
            pltpu.VMEM((PW_B,), jnp.int32),
            pltpu.VMEM((PW_B,), jnp.int32),
            pltpu.VMEM((CB, H), jnp.float32),
            pltpu.VMEM((CB, H), jnp.float32),
            pltpu.SemaphoreType.DMA,
            pltpu.SemaphoreType.DMA,
        ],
    )
    def k(a_ref, src_ref, ia_hbm, im_hbm, out_ref, ia_v, im_v, a_v, m_v, sa, sm):
        w = lax.axis_index("s") * 2 + lax.axis_index("c")
        pltpu.sync_copy(ia_hbm.at[pl.ds(w * PW_B, PW_B)], ia_v)
        pltpu.sync_copy(im_hbm.at[pl.ds(w * PW_B, PW_B)], im_v)

        def step(s, carry):
            ca = pltpu.make_async_copy(a_ref.at[ia_v.at[pl.ds(s * CB, CB)]], a_v, sa)
            cm = pltpu.make_async_copy(src_ref.at[im_v.at[pl.ds(s * CB, CB)]], m_v, sm)
            ca.start()
            cm.start()
            ca.wait()
            cm.wait()

            def row(r, c):
                for j in range(H // 16):
                    m = m_v[r, pl.ds(j * 16, 16)]
                    if do_relu:
                        m = jnp.maximum(m, 0.0)
                    a_v[r, pl.ds(j * 16, 16)] = a_v[r, pl.ds(j * 16, 16)] - m
                return c

            lax.fori_loop(0, CB, row, 0)
            pltpu.sync_copy(a_v, out_ref.at[pl.ds(w * PW_B + s * CB, CB)])
            return carry

        lax.fori_loop(0, SB, step, 0)

    return k(a_tab, src, b2a_pad, b2revb_pad)


def _mm_wi(f_bonds, w_i):
    blk = 512
    grid = (E + blk - 1) // blk

    def body(x_ref, w_ref, o_ref):
        o_ref[...] = jnp.dot(x_ref[...], w_ref[...],
                             preferred_element_type=jnp.float32)

    return pl.pallas_call(
        body,
        grid=(grid,),
        in_specs=[pl.BlockSpec((blk, BF), lambda g: (g, 0)),
                  pl.BlockSpec((BF, H), lambda g: (0, 0))],
        out_specs=pl.BlockSpec((blk, H), lambda g: (g, 0)),
        out_shape=jax.ShapeDtypeStruct((E, H), jnp.float32),
    )(f_bonds, w_i)


def _update(pre, inp, w_h):
    blk = 512
    grid = (E + blk - 1) // blk

    def body(p_ref, i_ref, w_ref, o_ref):
        acc = jnp.dot(p_ref[...], w_ref[...],
                      preferred_element_type=jnp.float32)
        o_ref[...] = jnp.maximum(i_ref[...] + acc, 0.0)

    return pl.pallas_call(
        body,
        grid=(grid,),
        in_specs=[pl.BlockSpec((blk, H), lambda g: (g, 0)),
                  pl.BlockSpec((blk, H), lambda g: (g, 0)),
                  pl.BlockSpec((H, H), lambda g: (0, 0))],
        out_specs=pl.BlockSpec((blk, H), lambda g: (g, 0)),
        out_shape=jax.ShapeDtypeStruct((E2, H), jnp.float32),
    )(pre, inp, w_h)


def _atom_head(f_atoms, a_fin, wo1, wo2, b_o2):
    blk = 512
    grid = (NA + blk - 1) // blk

    def body(x_ref, a_ref, w1_ref, w2_ref, b_ref, o_ref):
        acc = jnp.dot(x_ref[...], w1_ref[...],
                      preferred_element_type=jnp.float32)
        acc = acc + jnp.dot(a_ref[...], w2_ref[...],
                            preferred_element_type=jnp.float32)
        o_ref[...] = jnp.maximum(acc + b_ref[...], 0.0)

    return pl.pallas_call(
        body,
        grid=(grid,),
        in_specs=[pl.BlockSpec((blk, AF), lambda g: (g, 0)),
                  pl.BlockSpec((blk, H), lambda g: (g, 0)),
                  pl.BlockSpec((AF, H), lambda g: (0, 0)),
                  pl.BlockSpec((H, H), lambda g: (0, 0)),
                  pl.BlockSpec((1, H), lambda g: (0, 0))],
        out_specs=pl.BlockSpec((blk, H), lambda g: (g, 0)),
        out_shape=jax.ShapeDtypeStruct((NA, H), jnp.float32),
    )(f_atoms, a_fin, wo1, wo2, b_o2)


def _pair_stats(m2):
    grid = NPAIR // PB

    def body(m_ref, p_ref, s_ref, q_ref, x1_v, x2_v, sem1, sem2, acc_s, acc_q):
        g = pl.program_id(0)
        c1 = pltpu.make_async_copy(
            m_ref.at[pl.ds(PB * g, PB), 1, :], x1_v, sem1)
        c2 = pltpu.make_async_copy(
            m_ref.at[pl.ds(PB * g + 1, PB), 0, :], x2_v, sem2)
        c1.start()
        c2.start()
        c1.wait()
        c2.wait()
        p = (x1_v[...] + x2_v[...]) * 0.5
        p_ref[...] = p
        ps = jnp.sum(p, axis=0, keepdims=True)
        pq = jnp.sum(p * p, axis=0, keepdims=True)

        @pl.when(g == 0)
        def _():
            acc_s[...] = ps
            acc_q[...] = pq

        @pl.when(g > 0)
        def _():
            acc_s[...] = acc_s[...] + ps
            acc_q[...] = acc_q[...] + pq

        @pl.when(g == grid - 1)
        def _():
            s_ref[...] = acc_s[...]
            q_ref[...] = acc_q[...]

    return pl.pallas_call(
        body,
        grid=(grid,),
        in_specs=[pl.BlockSpec(memory_space=pl.ANY)],
        out_specs=[pl.BlockSpec((PB, H), lambda g: (g, 0)),
                   pl.BlockSpec((1, H), lambda g: (0, 0)),
                   pl.BlockSpec((1, H), lambda g: (0, 0))],
        out_shape=[jax.ShapeDtypeStruct((NPAIR, H), jnp.float32),
                   jax.ShapeDtypeStruct((1, H), jnp.float32),
                   jax.ShapeDtypeStruct((1, H), jnp.float32)],
        scratch_shapes=[pltpu.VMEM((PB, H), jnp.float32),
                        pltpu.VMEM((PB, H), jnp.float32),
                        pltpu.SemaphoreType.DMA,
                        pltpu.SemaphoreType.DMA,
                        pltpu.VMEM((1, H), jnp.float32),
                        pltpu.VMEM((1, H), jnp.float32)],
    )(m2)


def _bn_apply(x, s1, q1, bn_w2, bn_b2, n):
    rows = x.shape[0]
    blk = 2000
    grid = rows // blk

    def body(x_ref, s_ref, q_ref, w_ref, b_ref, o_ref):
        mean = s_ref[...] / n
        var = q_ref[...] / n - mean * mean
        inv = lax.rsqrt(var + EPS) * w_ref[...]
        o_ref[...] = (x_ref[...] - mean) * inv + b_ref[...]

    return pl.pallas_call(
        body,
        grid=(grid,),
        in_specs=[pl.BlockSpec((blk, H), lambda g: (g, 0)),
                  pl.BlockSpec((1, H), lambda g: (0, 0)),
                  pl.BlockSpec((1, H), lambda g: (0, 0)),
                  pl.BlockSpec((1, H), lambda g: (0, 0)),
                  pl.BlockSpec((1, H), lambda g: (0, 0))],
        out_specs=pl.BlockSpec((blk, H), lambda g: (g, 0)),
        out_shape=jax.ShapeDtypeStruct((rows, H), jnp.float32),
    )(x, s1, q1, bn_w2, bn_b2)


def _bn_full(x, bn_w2, bn_b2):
    rows = x.shape[0]

    def body(x_ref, w_ref, b_ref, o_ref):
        xx = x_ref[...]
        mean = jnp.sum(xx, axis=0, keepdims=True) / rows
        var = jnp.sum(xx * xx, axis=0, keepdims=True) / rows - mean * mean
        o_ref[...] = (xx - mean) * lax.rsqrt(var + EPS) * w_ref[...] + b_ref[...]

    return pl.pallas_call(
        body,
        in_specs=[pl.BlockSpec((rows, H), lambda: (0, 0)),
                  pl.BlockSpec((1, H), lambda: (0, 0)),
                  pl.BlockSpec((1, H), lambda: (0, 0))],
        out_specs=pl.BlockSpec((rows, H), lambda: (0, 0)),
        out_shape=jax.ShapeDtypeStruct((rows, H), jnp.float32),
    )(x, bn_w2, bn_b2)


def kernel(f_atoms, f_bonds, W_i, W_h, W_o, b_o, bn_w, bn_b, a2b, b2a, b2revb):
    a2b = a2b.astype(jnp.int32)
    b2a = b2a.astype(jnp.int32)
    b2revb = b2revb.astype(jnp.int32)

    a2b_flat = jnp.pad(a2b.reshape(-1), (0, A_PAD * NB - NA * NB))
    b2a_p = jnp.pad(b2a, (0, E_PAD - E))
    b2revb_p = jnp.pad(b2revb, (0, E_PAD - E))
    wo1 = W_o[:AF]
    wo2 = W_o[AF:]
    b_o2 = b_o.reshape(1, H)
    bn_w2 = bn_w.reshape(1, H)
    bn_b2 = bn_b.reshape(1, H)

    inp = _mm_wi(f_bonds, W_i)

    src = inp
    relu_flag = True
    for _ in range(2):
        a_tab = _gather_sum_sc(src, a2b_flat, relu_flag)
        pre = _pre_sc(a_tab, src, b2a_p, b2revb_p, relu_flag)
        src = _update(pre, inp, W_h)
        relu_flag = False

    a_fin = _gather_sum_sc(src, a2b_flat, False)
    ah = _atom_head(f_atoms, a_fin, wo1, wo2, b_o2)

    pair_m, s1, q1 = _pair_stats(src.reshape(E2 // 2, 2, H))
    bonds_v = _bn_apply(pair_m, s1, q1, bn_w2, bn_b2, float(NPAIR))
    atoms_v = _bn_full(ah[1:NA], bn_w2, bn_b2)
    return atoms_v, bonds_v
